# baseline (speedup 1.0000x reference)
import functools

import jax
import jax.numpy as jnp
from jax import lax
from jax.experimental import pallas as pl
from jax.experimental.pallas import tpu as pltpu
from jax.experimental.pallas import tpu_sc as plsc

_NUM_CLASS = 7
_N = 16 * 512 * 512
_NW = 32
_PER_W = _N // _NW
_IPC = 2
_NCHUNK = 16 // _IPC
_ROWS_W = 512 // _NW
_CHUNK = _IPC * _ROWS_W * 512
_UNROLL = 32
_NROWS = 32


def _sc_body(pred_hbm, ref_hbm, lbl_hbm, out_hbm, pbuf, rbuf, lbuf, acc, sems):
    wid = lax.axis_index("s") * 2 + lax.axis_index("c")
    r0 = wid * _ROWS_W

    for i in range(_NROWS):
        acc[i] = jnp.zeros((16,), jnp.float32)

    def copies(ci, b):
        i0 = ci * _IPC
        return [
            pltpu.make_async_copy(
                pred_hbm.at[pl.ds(i0, _IPC), pl.ds(r0, _ROWS_W), :],
                pbuf.at[b], sems.at[b]),
            pltpu.make_async_copy(
                ref_hbm.at[pl.ds(i0, _IPC), pl.ds(r0, _ROWS_W), :],
                rbuf.at[b], sems.at[b]),
            pltpu.make_async_copy(
                lbl_hbm.at[pl.ds(i0, _IPC), pl.ds(r0, _ROWS_W), :],
                lbuf.at[b], sems.at[b]),
        ]

    for c in copies(0, 0):
        c.start()

    for ci in range(_NCHUNK):
        b = ci % 2
        if ci + 1 < _NCHUNK:
            for c in copies(ci + 1, 1 - b):
                c.start()
        for c in copies(ci, b):
            c.wait()

        @plsc.parallel_loop(0, _CHUNK // 16, unroll=_UNROLL)
        def vec_body(j):
            lane = lax.iota(jnp.int32, 16)
            ones = jnp.ones((16,), jnp.float32)
            zerof = jnp.zeros((16,), jnp.float32)
            c8 = jnp.full((16,), 8, jnp.int32)
            c24 = jnp.full((16,), 24, jnp.int32)
            i2 = lax.shift_right_logical(j, 9)
            row = lax.bitwise_and(lax.shift_right_logical(j, 5), 15)
            col = lax.mul(lax.bitwise_and(j, 31), 16)
            p = pbuf[b, i2, row, pl.ds(col, 16)]
            r = rbuf[b, i2, row, pl.ds(col, 16)]
            l = lbuf[b, i2, row, pl.ds(col, 16)]
            d = p - r
            neg = (d < zerof).astype(jnp.int32)
            l4 = l * jnp.full((16,), 4, jnp.int32)
            one_i = jnp.full((16,), 1, jnp.int32)
            c3 = jnp.full((16,), 3, jnp.int32)
            plsc.addupdate_scatter(acc, [l4, lane], d * d)
            plsc.addupdate_scatter(acc, [l4 + one_i + neg, lane], d)
            plsc.addupdate_scatter(acc, [l4 + c3, lane], ones)

    pltpu.sync_copy(acc, out_hbm.at[wid])


def _hist_body(l_ref, out_ref):
    i = pl.program_id(0)
    x = l_ref[0]
    rows = lax.broadcasted_iota(jnp.int32, (8, 128), 0)
    acc = jnp.zeros((8, 128), jnp.float32)
    for c in range(_NUM_CLASS):
        s = jnp.sum((x == c).astype(jnp.float32))
        acc = acc + jnp.where(rows == c, s, 0.0)

    @pl.when(i == 0)
    def _():
        out_ref[...] = jnp.zeros((8, 128), jnp.float32)

    out_ref[...] += acc


def _finish_body(part_ref, hist_ref, stats_ref, cnt_ref):
    x = part_ref[...]
    s = jnp.sum(jnp.sum(x, axis=0), axis=2)
    sumsq = s[:, 0:1]
    pos = s[:, 1:2]
    neg = s[:, 2:3]
    cnt = hist_ref[:, 0:1]
    sum_abs = pos - neg
    sum_me = pos + neg
    safe = jnp.maximum(cnt, 1.0)
    has = cnt > 0.0
    rmse_t = jnp.where(has, jnp.sqrt(sumsq / safe) * cnt, 0.0)
    mae_t = jnp.where(has, sum_abs, 0.0)
    me_t = jnp.where(has, sum_me, 0.0)
    full = jnp.concatenate([rmse_t, mae_t, me_t], axis=1)
    stats_ref[...] = full[:_NUM_CLASS, :]
    cnt_ref[...] = cnt[:_NUM_CLASS, :]


@jax.jit
def kernel(pred, ref, buildhir):
    p = pred
    r = ref
    l = buildhir.astype(jnp.int32)

    mesh = plsc.VectorSubcoreMesh(core_axis_name="c", subcore_axis_name="s")
    sc = pl.kernel(
        _sc_body,
        out_type=jax.ShapeDtypeStruct((_NW, _NROWS, 16), jnp.float32),
        mesh=mesh,
        scratch_types=[
            pltpu.VMEM((2, _IPC, _ROWS_W, 512), jnp.float32),
            pltpu.VMEM((2, _IPC, _ROWS_W, 512), jnp.float32),
            pltpu.VMEM((2, _IPC, _ROWS_W, 512), jnp.int32),
            pltpu.VMEM((_NROWS, 16), jnp.float32),
            pltpu.SemaphoreType.DMA((2,)),
        ],
        compiler_params=pltpu.CompilerParams(
            needs_layout_passes=False, use_tc_tiling_on_sc=True),
    )
    partials = sc(p, r, l)

    hist = pl.pallas_call(
        _hist_body,
        grid=(16,),
        in_specs=[pl.BlockSpec((1, 512, 512), lambda i: (i, 0, 0))],
        out_specs=pl.BlockSpec((8, 128), lambda i: (0, 0)),
        out_shape=jax.ShapeDtypeStruct((8, 128), jnp.float32),
    )(l)

    z = partials.reshape(_NW, 8, 4, 16)
    stats, count = pl.pallas_call(
        _finish_body,
        out_shape=[
            jax.ShapeDtypeStruct((_NUM_CLASS, 3), jnp.float32),
            jax.ShapeDtypeStruct((_NUM_CLASS, 1), jnp.float32),
        ],
    )(z, hist)
    return stats, count


# speedup vs baseline: 1.0804x; 1.0804x over previous
import functools

import jax
import jax.numpy as jnp
from jax import lax
from jax.experimental import pallas as pl
from jax.experimental.pallas import tpu as pltpu
from jax.experimental.pallas import tpu_sc as plsc

_NUM_CLASS = 7
_N = 16 * 512 * 512
_NW = 32
_PER_W = _N // _NW
_IPC = 2
_NCHUNK = 16 // _IPC
_ROWS_W = 512 // _NW
_CHUNK = _IPC * _ROWS_W * 512
_UNROLL = 32
_NROWS = 32


def _sc_body(pred_hbm, ref_hbm, lbl_hbm, out_hbm, pbuf, rbuf, lbuf, acc, sems):
    wid = lax.axis_index("s") * 2 + lax.axis_index("c")
    r0 = wid * _ROWS_W

    for i in range(_NROWS):
        acc[i] = jnp.zeros((16,), jnp.float32)

    def copies(ci, b):
        i0 = ci * _IPC
        return [
            pltpu.make_async_copy(
                pred_hbm.at[pl.ds(i0, _IPC), pl.ds(r0, _ROWS_W), :],
                pbuf.at[b], sems.at[b]),
            pltpu.make_async_copy(
                ref_hbm.at[pl.ds(i0, _IPC), pl.ds(r0, _ROWS_W), :],
                rbuf.at[b], sems.at[b]),
            pltpu.make_async_copy(
                lbl_hbm.at[pl.ds(i0, _IPC), pl.ds(r0, _ROWS_W), :],
                lbuf.at[b], sems.at[b]),
        ]

    for c in copies(0, 0):
        c.start()

    for ci in range(_NCHUNK):
        b = ci % 2
        if ci + 1 < _NCHUNK:
            for c in copies(ci + 1, 1 - b):
                c.start()
        for c in copies(ci, b):
            c.wait()

        @plsc.parallel_loop(0, _CHUNK // 16, unroll=_UNROLL)
        def vec_body(j):
            lane = lax.iota(jnp.int32, 16)
            ones = jnp.ones((16,), jnp.float32)
            zerof = jnp.zeros((16,), jnp.float32)
            c8 = jnp.full((16,), 8, jnp.int32)
            c24 = jnp.full((16,), 24, jnp.int32)
            i2 = lax.shift_right_logical(j, 9)
            row = lax.bitwise_and(lax.shift_right_logical(j, 5), 15)
            col = lax.mul(lax.bitwise_and(j, 31), 16)
            p = pbuf[b, i2, row, pl.ds(col, 16)]
            r = rbuf[b, i2, row, pl.ds(col, 16)]
            l = lbuf[b, i2, row, pl.ds(col, 16)]
            d = p - r
            neg = (d < zerof).astype(jnp.int32)
            l4 = l * jnp.full((16,), 4, jnp.int32)
            one_i = jnp.full((16,), 1, jnp.int32)
            c3 = jnp.full((16,), 3, jnp.int32)
            plsc.addupdate_scatter(acc, [l4, lane], d * d)
            plsc.addupdate_scatter(acc, [l4 + one_i + neg, lane], d)

    pltpu.sync_copy(acc, out_hbm.at[wid])


def _hist_body(l_ref, out_ref):
    i = pl.program_id(0)
    x = l_ref[0]
    rows = lax.broadcasted_iota(jnp.int32, (8, 128), 0)
    acc = jnp.zeros((8, 128), jnp.float32)
    for c in range(_NUM_CLASS):
        s = jnp.sum((x == c).astype(jnp.float32))
        acc = acc + jnp.where(rows == c, s, 0.0)

    @pl.when(i == 0)
    def _():
        out_ref[...] = jnp.zeros((8, 128), jnp.float32)

    out_ref[...] += acc


def _finish_body(part_ref, hist_ref, stats_ref, cnt_ref):
    x = part_ref[...]
    s = jnp.sum(jnp.sum(x, axis=0), axis=2)
    sumsq = s[:, 0:1]
    pos = s[:, 1:2]
    neg = s[:, 2:3]
    cnt = hist_ref[:, 0:1]
    sum_abs = pos - neg
    sum_me = pos + neg
    safe = jnp.maximum(cnt, 1.0)
    has = cnt > 0.0
    rmse_t = jnp.where(has, jnp.sqrt(sumsq / safe) * cnt, 0.0)
    mae_t = jnp.where(has, sum_abs, 0.0)
    me_t = jnp.where(has, sum_me, 0.0)
    full = jnp.concatenate([rmse_t, mae_t, me_t], axis=1)
    stats_ref[...] = full[:_NUM_CLASS, :]
    cnt_ref[...] = cnt[:_NUM_CLASS, :]


@jax.jit
def kernel(pred, ref, buildhir):
    p = pred
    r = ref
    l = buildhir.astype(jnp.int32)

    mesh = plsc.VectorSubcoreMesh(core_axis_name="c", subcore_axis_name="s")
    sc = pl.kernel(
        _sc_body,
        out_type=jax.ShapeDtypeStruct((_NW, _NROWS, 16), jnp.float32),
        mesh=mesh,
        scratch_types=[
            pltpu.VMEM((2, _IPC, _ROWS_W, 512), jnp.float32),
            pltpu.VMEM((2, _IPC, _ROWS_W, 512), jnp.float32),
            pltpu.VMEM((2, _IPC, _ROWS_W, 512), jnp.int32),
            pltpu.VMEM((_NROWS, 16), jnp.float32),
            pltpu.SemaphoreType.DMA((2,)),
        ],
        compiler_params=pltpu.CompilerParams(
            needs_layout_passes=False, use_tc_tiling_on_sc=True),
    )
    partials = sc(p, r, l)

    hist = pl.pallas_call(
        _hist_body,
        grid=(16,),
        in_specs=[pl.BlockSpec((1, 512, 512), lambda i: (i, 0, 0))],
        out_specs=pl.BlockSpec((8, 128), lambda i: (0, 0)),
        out_shape=jax.ShapeDtypeStruct((8, 128), jnp.float32),
    )(l)

    z = partials.reshape(_NW, 8, 4, 16)
    stats, count = pl.pallas_call(
        _finish_body,
        out_shape=[
            jax.ShapeDtypeStruct((_NUM_CLASS, 3), jnp.float32),
            jax.ShapeDtypeStruct((_NUM_CLASS, 1), jnp.float32),
        ],
    )(z, hist)
    return stats, count
